# baseline (speedup 1.0000x reference)
import jax
import jax.numpy as jnp
from jax import lax
from jax.experimental import pallas as pl
from jax.experimental.pallas import tpu as pltpu

_B = 16
_S = 512
_K = 10
_NUM_EXP = 500
_OVER = 5.0
_NB = 5
_DTMAX = 5.0
_R = _B * _S
_BLK_R = 512
_CHUNK = 128
_NCHUNK = 4
_WLANES = 128
_BIG = 3.0e38


def _softplus(z):
    return jnp.maximum(z, 0.0) + jnp.log1p(jnp.exp(-jnp.abs(z)))


_ROT = ((13, 15, 26, 6), (17, 29, 16, 24))


def _threefry_hash(k0, k1, x1):
    ks2 = k0 ^ k1 ^ jnp.uint32(0x1BD11BDA)
    inj = ((k1, ks2, 1), (ks2, k0, 2), (k0, k1, 3), (k1, ks2, 4), (ks2, k0, 5))
    x0 = jnp.zeros_like(x1) + k0
    x1 = x1 + k1
    for g in range(5):
        for r in _ROT[g % 2]:
            x0 = x0 + x1
            x1 = (x1 << jnp.uint32(r)) | (x1 >> jnp.uint32(32 - r))
            x1 = x0 ^ x1
        a, b, c = inj[g]
        x0 = x0 + a
        x1 = x1 + (b + jnp.uint32(c))
    return x0 ^ x1


def _bits_to_unif(bits):
    f = lax.bitcast_convert_type((bits >> jnp.uint32(9)) | jnp.uint32(0x3F800000),
                                 jnp.float32)
    return f - 1.0


def _body(ts_ref, wbv_ref, wbs_ref, keys_ref, out_ref):
    wv = wbv_ref[0:1, :]
    bv = wbv_ref[1:2, :]
    subf = lax.broadcasted_iota(jnp.int32, (8, _WLANES), 0).astype(jnp.float32)
    lane = lax.broadcasted_iota(jnp.int32, (8, _WLANES), 1)
    steps = 1e-05 + subf * ((_DTMAX - 1e-05) / (_NB - 1))
    lam = _softplus(bv - wv * steps)
    lam = jnp.where(lane < _K, lam, 0.0)
    row = jnp.sum(lam, axis=1, keepdims=True)
    subi = lax.broadcasted_iota(jnp.int32, (8, 1), 0)
    row = jnp.where(subi < _NB, row, -jnp.inf)
    bound = jnp.max(row) * _OVER
    inv_bound = 1.0 / (bound + 1e-09)

    ke0 = keys_ref[0, 0]
    ke1 = keys_ref[0, 1]
    ku0 = keys_ref[0, 2]
    ku1 = keys_ref[0, 3]

    i = pl.program_id(0)
    rowg = i * _BLK_R + lax.broadcasted_iota(jnp.int32, (_BLK_R, _CHUNK), 0)
    base_idx = rowg * _NUM_EXP
    col_iota = lax.broadcasted_iota(jnp.int32, (_BLK_R, _CHUNK), 1)
    ntri = -(lax.broadcasted_iota(jnp.int32, (_CHUNK, _CHUNK), 0)
             <= lax.broadcasted_iota(jnp.int32, (_CHUNK, _CHUNK), 1)
             ).astype(jnp.float32)

    def cond(state):
        j, carry, val = state
        return (j < _NCHUNK) & jnp.logical_not(jnp.all(val < _BIG))

    _LOG2E = 1.4426950408889634
    sk = [(wbs_ref[0, k] * inv_bound) * _LOG2E for k in range(_K)]
    bk = [wbs_ref[1, k] * _LOG2E for k in range(_K)]
    bound2 = bound * _LOG2E

    def chunk_body(state):
        j, carry, val = state
        colg = j * _CHUNK + col_iota
        idx = (base_idx + colg).astype(jnp.uint32)
        u = _bits_to_unif(_threefry_hash(ke0, ke1, idx))
        unif = _bits_to_unif(_threefry_hash(ku0, ku1, idx))
        lanemask = colg < _NUM_EXP
        x = jnp.where(lanemask, jnp.log(u + 1e-09), 0.0)
        cs = jnp.dot(x, ntri, preferred_element_type=jnp.float32)
        c_un = carry + cs
        prod = jnp.ones_like(c_un)
        for k in range(_K):
            prod = prod * (1.0 + jnp.exp2(bk[k] - sk[k] * c_un))
        acc = (jnp.exp2(unif * bound2) < prod) & lanemask
        cmin = jnp.min(jnp.where(acc, c_un, _BIG), axis=1, keepdims=True)
        return j + 1, carry + cs[:, _CHUNK - 1:_CHUNK], jnp.minimum(val, cmin)

    idx0 = (base_idx + col_iota).astype(jnp.uint32)
    u0 = _bits_to_unif(_threefry_hash(ke0, ke1, idx0))
    unif0 = _bits_to_unif(_threefry_hash(ku0, ku1, idx0))
    x0 = jnp.log(u0 + 1e-09)
    cs0 = jnp.dot(x0, ntri, preferred_element_type=jnp.float32)
    prod0 = jnp.ones_like(cs0)
    for k in range(_K):
        prod0 = prod0 * (1.0 + jnp.exp2(bk[k] - sk[k] * cs0))
    acc0 = jnp.exp2(unif0 * bound2) < prod0
    val0 = jnp.min(jnp.where(acc0, cs0, _BIG), axis=1, keepdims=True)

    init = (jnp.int32(1), cs0[:, _CHUNK - 1:_CHUNK], val0)
    _, _, val = lax.while_loop(cond, chunk_body, init)

    res = jnp.where(val < _BIG,
                    jnp.minimum(ts_ref[...] + val * inv_bound, 100000.0),
                    jnp.float32(_DTMAX))
    out_ref[...] = res


def kernel(time_seq, time_delta_seq, event_seq, w, b):
    kroot = jax.random.key(1)
    ke, ku = jax.random.split(kroot)
    kde = jax.random.key_data(ke).astype(jnp.uint32)
    kdu = jax.random.key_data(ku).astype(jnp.uint32)
    keys = jnp.concatenate([kde, kdu]).reshape(1, 4)

    ts = time_seq.reshape(_R, 1)
    wb = jnp.zeros((2, _WLANES), jnp.float32)
    wb = wb.at[0, :_K].set(w).at[1, :_K].set(b)

    grid = (_R // _BLK_R,)
    res = pl.pallas_call(
        _body,
        grid=grid,
        in_specs=[
            pl.BlockSpec((_BLK_R, 1), lambda i: (i, 0)),
            pl.BlockSpec((2, _WLANES), lambda i: (0, 0)),
            pl.BlockSpec(memory_space=pltpu.SMEM),
            pl.BlockSpec(memory_space=pltpu.SMEM),
        ],
        out_specs=pl.BlockSpec((_BLK_R, 1), lambda i: (i, 0)),
        out_shape=jax.ShapeDtypeStruct((_R, 1), jnp.float32),
    )(ts, wb, wb, keys)

    res = res.reshape(_B, _S, 1)
    weights = jnp.ones_like(res)
    return (res, weights)


# speedup vs baseline: 1.2410x; 1.2410x over previous
import jax
import jax.numpy as jnp
from jax import lax
from jax.experimental import pallas as pl
from jax.experimental.pallas import tpu as pltpu

_B = 16
_S = 512
_K = 10
_NUM_EXP = 500
_OVER = 5.0
_NB = 5
_DTMAX = 5.0
_R = _B * _S
_BLK_R = 256
_CHUNK = 128
_NCHUNK = 4
_WLANES = 128
_BIG = 3.0e38


def _softplus(z):
    return jnp.maximum(z, 0.0) + jnp.log1p(jnp.exp(-jnp.abs(z)))


_ROT = ((13, 15, 26, 6), (17, 29, 16, 24))


def _threefry_hash(k0, k1, x1):
    ks2 = k0 ^ k1 ^ jnp.uint32(0x1BD11BDA)
    inj = ((k1, ks2, 1), (ks2, k0, 2), (k0, k1, 3), (k1, ks2, 4), (ks2, k0, 5))
    x0 = jnp.zeros_like(x1) + k0
    x1 = x1 + k1
    for g in range(5):
        for r in _ROT[g % 2]:
            x0 = x0 + x1
            x1 = (x1 << jnp.uint32(r)) | (x1 >> jnp.uint32(32 - r))
            x1 = x0 ^ x1
        a, b, c = inj[g]
        x0 = x0 + a
        x1 = x1 + (b + jnp.uint32(c))
    return x0 ^ x1


def _bits_to_unif(bits):
    f = lax.bitcast_convert_type((bits >> jnp.uint32(9)) | jnp.uint32(0x3F800000),
                                 jnp.float32)
    return f - 1.0


def _body(ts_ref, wbv_ref, wbs_ref, keys_ref, out_ref):
    wv = wbv_ref[0:1, :]
    bv = wbv_ref[1:2, :]
    subf = lax.broadcasted_iota(jnp.int32, (8, _WLANES), 0).astype(jnp.float32)
    lane = lax.broadcasted_iota(jnp.int32, (8, _WLANES), 1)
    steps = 1e-05 + subf * ((_DTMAX - 1e-05) / (_NB - 1))
    lam = _softplus(bv - wv * steps)
    lam = jnp.where(lane < _K, lam, 0.0)
    row = jnp.sum(lam, axis=1, keepdims=True)
    subi = lax.broadcasted_iota(jnp.int32, (8, 1), 0)
    row = jnp.where(subi < _NB, row, -jnp.inf)
    bound = jnp.max(row) * _OVER
    inv_bound = 1.0 / (bound + 1e-09)

    ke0 = keys_ref[0, 0]
    ke1 = keys_ref[0, 1]
    ku0 = keys_ref[0, 2]
    ku1 = keys_ref[0, 3]

    i = pl.program_id(0)
    rowg = i * _BLK_R + lax.broadcasted_iota(jnp.int32, (_BLK_R, _CHUNK), 0)
    base_idx = rowg * _NUM_EXP
    col_iota = lax.broadcasted_iota(jnp.int32, (_BLK_R, _CHUNK), 1)
    ntri = -(lax.broadcasted_iota(jnp.int32, (_CHUNK, _CHUNK), 0)
             <= lax.broadcasted_iota(jnp.int32, (_CHUNK, _CHUNK), 1)
             ).astype(jnp.float32)

    def cond(state):
        j, carry, val = state
        return (j < _NCHUNK) & jnp.logical_not(jnp.all(val < _BIG))

    _LOG2E = 1.4426950408889634
    sk = [(wbs_ref[0, k] * inv_bound) * _LOG2E for k in range(_K)]
    bk = [wbs_ref[1, k] * _LOG2E for k in range(_K)]
    bound2 = bound * _LOG2E

    def chunk_body(state):
        j, carry, val = state
        colg = j * _CHUNK + col_iota
        idx = (base_idx + colg).astype(jnp.uint32)
        u = _bits_to_unif(_threefry_hash(ke0, ke1, idx))
        unif = _bits_to_unif(_threefry_hash(ku0, ku1, idx))
        lanemask = colg < _NUM_EXP
        x = jnp.where(lanemask, jnp.log(u + 1e-09), 0.0)
        cs = jnp.dot(x, ntri, preferred_element_type=jnp.float32)
        c_un = carry + cs
        prod = jnp.ones_like(c_un)
        for k in range(_K):
            prod = prod * (1.0 + jnp.exp2(bk[k] - sk[k] * c_un))
        acc = (jnp.exp2(unif * bound2) < prod) & lanemask
        cmin = jnp.min(jnp.where(acc, c_un, _BIG), axis=1, keepdims=True)
        return j + 1, carry + cs[:, _CHUNK - 1:_CHUNK], jnp.minimum(val, cmin)

    idx0 = (base_idx + col_iota).astype(jnp.uint32)
    u0 = _bits_to_unif(_threefry_hash(ke0, ke1, idx0))
    unif0 = _bits_to_unif(_threefry_hash(ku0, ku1, idx0))
    x0 = jnp.log(u0 + 1e-09)
    cs0 = jnp.dot(x0, ntri, preferred_element_type=jnp.float32)
    prod0 = jnp.ones_like(cs0)
    for k in range(_K):
        prod0 = prod0 * (1.0 + jnp.exp2(bk[k] - sk[k] * cs0))
    acc0 = jnp.exp2(unif0 * bound2) < prod0
    val0 = jnp.min(jnp.where(acc0, cs0, _BIG), axis=1, keepdims=True)

    init = (jnp.int32(1), cs0[:, _CHUNK - 1:_CHUNK], val0)
    _, _, val = lax.while_loop(cond, chunk_body, init)

    res = jnp.where(val < _BIG,
                    jnp.minimum(ts_ref[...] + val * inv_bound, 100000.0),
                    jnp.float32(_DTMAX))
    out_ref[...] = res


def kernel(time_seq, time_delta_seq, event_seq, w, b):
    kroot = jax.random.key(1)
    ke, ku = jax.random.split(kroot)
    kde = jax.random.key_data(ke).astype(jnp.uint32)
    kdu = jax.random.key_data(ku).astype(jnp.uint32)
    keys = jnp.concatenate([kde, kdu]).reshape(1, 4)

    ts = time_seq.reshape(_R, 1)
    wb = jnp.zeros((2, _WLANES), jnp.float32)
    wb = wb.at[0, :_K].set(w).at[1, :_K].set(b)

    grid = (_R // _BLK_R,)
    res = pl.pallas_call(
        _body,
        grid=grid,
        in_specs=[
            pl.BlockSpec((_BLK_R, 1), lambda i: (i, 0)),
            pl.BlockSpec((2, _WLANES), lambda i: (0, 0)),
            pl.BlockSpec(memory_space=pltpu.SMEM),
            pl.BlockSpec(memory_space=pltpu.SMEM),
        ],
        out_specs=pl.BlockSpec((_BLK_R, 1), lambda i: (i, 0)),
        out_shape=jax.ShapeDtypeStruct((_R, 1), jnp.float32),
    )(ts, wb, wb, keys)

    res = res.reshape(_B, _S, 1)
    weights = jnp.ones_like(res)
    return (res, weights)


# speedup vs baseline: 1.2469x; 1.0047x over previous
import jax
import jax.numpy as jnp
from jax import lax
from jax.experimental import pallas as pl
from jax.experimental.pallas import tpu as pltpu

_B = 16
_S = 512
_K = 10
_NUM_EXP = 500
_OVER = 5.0
_NB = 5
_DTMAX = 5.0
_R = _B * _S
_BLK_R = 256
_CHUNK = 128
_NCHUNK = 4
_WLANES = 128
_BIG = 3.0e38


def _softplus(z):
    return jnp.maximum(z, 0.0) + jnp.log1p(jnp.exp(-jnp.abs(z)))


_ROT = ((13, 15, 26, 6), (17, 29, 16, 24))


def _threefry_hash(k0, k1, x1):
    ks2 = k0 ^ k1 ^ jnp.uint32(0x1BD11BDA)
    inj = ((k1, ks2, 1), (ks2, k0, 2), (k0, k1, 3), (k1, ks2, 4), (ks2, k0, 5))
    x0 = jnp.zeros_like(x1) + k0
    x1 = x1 + k1
    for g in range(5):
        for r in _ROT[g % 2]:
            x0 = x0 + x1
            x1 = (x1 << jnp.uint32(r)) | (x1 >> jnp.uint32(32 - r))
            x1 = x0 ^ x1
        a, b, c = inj[g]
        x0 = x0 + a
        x1 = x1 + (b + jnp.uint32(c))
    return x0 ^ x1


def _bits_to_unif(bits):
    f = lax.bitcast_convert_type((bits >> jnp.uint32(9)) | jnp.uint32(0x3F800000),
                                 jnp.float32)
    return f - 1.0


def _body(ts_ref, wbv_ref, wbs_ref, keys_ref, out_ref,
          ntri_ref, idxb_ref, bsc_ref):
    i = pl.program_id(0)

    @pl.when(i == 0)
    def _prep():
        wv = wbv_ref[0:1, :]
        bv = wbv_ref[1:2, :]
        subf = lax.broadcasted_iota(jnp.int32, (8, _WLANES), 0).astype(jnp.float32)
        lane = lax.broadcasted_iota(jnp.int32, (8, _WLANES), 1)
        steps = 1e-05 + subf * ((_DTMAX - 1e-05) / (_NB - 1))
        lam = _softplus(bv - wv * steps)
        lam = jnp.where(lane < _K, lam, 0.0)
        row = jnp.sum(lam, axis=1, keepdims=True)
        subi = lax.broadcasted_iota(jnp.int32, (8, 1), 0)
        row = jnp.where(subi < _NB, row, -jnp.inf)
        bound_v = jnp.max(row) * _OVER
        bsc_ref[0, 0] = bound_v
        bsc_ref[0, 1] = 1.0 / (bound_v + 1e-09)
        ntri_ref[...] = -(lax.broadcasted_iota(jnp.int32, (_CHUNK, _CHUNK), 0)
                          <= lax.broadcasted_iota(jnp.int32, (_CHUNK, _CHUNK), 1)
                          ).astype(jnp.float32)
        idxb_ref[...] = (lax.broadcasted_iota(jnp.int32, (_BLK_R, _CHUNK), 0)
                         * _NUM_EXP
                         + lax.broadcasted_iota(jnp.int32, (_BLK_R, _CHUNK), 1))

    bound = bsc_ref[0, 0]
    inv_bound = bsc_ref[0, 1]

    ke0 = keys_ref[0, 0]
    ke1 = keys_ref[0, 1]
    ku0 = keys_ref[0, 2]
    ku1 = keys_ref[0, 3]

    idxb = idxb_ref[...]
    ntri = ntri_ref[...]
    col_iota = lax.broadcasted_iota(jnp.int32, (_BLK_R, _CHUNK), 1)
    step_off = i * (_BLK_R * _NUM_EXP)

    def cond(state):
        j, carry, val = state
        return (j < _NCHUNK) & jnp.logical_not(jnp.all(val < _BIG))

    _LOG2E = 1.4426950408889634
    sk = [(wbs_ref[0, k] * inv_bound) * _LOG2E for k in range(_K)]
    bk = [wbs_ref[1, k] * _LOG2E for k in range(_K)]
    bound2 = bound * _LOG2E

    def chunk_body(state):
        j, carry, val = state
        colg = j * _CHUNK + col_iota
        idx = (idxb + (step_off + j * _CHUNK)).astype(jnp.uint32)
        u = _bits_to_unif(_threefry_hash(ke0, ke1, idx))
        unif = _bits_to_unif(_threefry_hash(ku0, ku1, idx))
        lanemask = colg < _NUM_EXP
        x = jnp.where(lanemask, jnp.log(u + 1e-09), 0.0)
        cs = jnp.dot(x, ntri, preferred_element_type=jnp.float32)
        c_un = carry + cs
        prod = jnp.ones_like(c_un)
        for k in range(_K):
            prod = prod * (1.0 + jnp.exp2(bk[k] - sk[k] * c_un))
        acc = (jnp.exp2(unif * bound2) < prod) & lanemask
        cmin = jnp.min(jnp.where(acc, c_un, _BIG), axis=1, keepdims=True)
        return j + 1, carry + cs[:, _CHUNK - 1:_CHUNK], jnp.minimum(val, cmin)

    idx0 = (idxb + step_off).astype(jnp.uint32)
    u0 = _bits_to_unif(_threefry_hash(ke0, ke1, idx0))
    unif0 = _bits_to_unif(_threefry_hash(ku0, ku1, idx0))
    x0 = jnp.log(u0 + 1e-09)
    cs0 = jnp.dot(x0, ntri, preferred_element_type=jnp.float32)
    prod0 = jnp.ones_like(cs0)
    for k in range(_K):
        prod0 = prod0 * (1.0 + jnp.exp2(bk[k] - sk[k] * cs0))
    acc0 = jnp.exp2(unif0 * bound2) < prod0
    val0 = jnp.min(jnp.where(acc0, cs0, _BIG), axis=1, keepdims=True)

    init = (jnp.int32(1), cs0[:, _CHUNK - 1:_CHUNK], val0)
    _, _, val = lax.while_loop(cond, chunk_body, init)

    res = jnp.where(val < _BIG,
                    jnp.minimum(ts_ref[...] + val * inv_bound, 100000.0),
                    jnp.float32(_DTMAX))
    out_ref[...] = res


def kernel(time_seq, time_delta_seq, event_seq, w, b):
    kroot = jax.random.key(1)
    ke, ku = jax.random.split(kroot)
    kde = jax.random.key_data(ke).astype(jnp.uint32)
    kdu = jax.random.key_data(ku).astype(jnp.uint32)
    keys = jnp.concatenate([kde, kdu]).reshape(1, 4)

    ts = time_seq.reshape(_R, 1)
    wb = jnp.zeros((2, _WLANES), jnp.float32)
    wb = wb.at[0, :_K].set(w).at[1, :_K].set(b)

    grid = (_R // _BLK_R,)
    res = pl.pallas_call(
        _body,
        grid=grid,
        in_specs=[
            pl.BlockSpec((_BLK_R, 1), lambda i: (i, 0)),
            pl.BlockSpec((2, _WLANES), lambda i: (0, 0)),
            pl.BlockSpec(memory_space=pltpu.SMEM),
            pl.BlockSpec(memory_space=pltpu.SMEM),
        ],
        out_specs=pl.BlockSpec((_BLK_R, 1), lambda i: (i, 0)),
        out_shape=jax.ShapeDtypeStruct((_R, 1), jnp.float32),
        scratch_shapes=[
            pltpu.VMEM((_CHUNK, _CHUNK), jnp.float32),
            pltpu.VMEM((_BLK_R, _CHUNK), jnp.int32),
            pltpu.SMEM((1, 2), jnp.float32),
        ],
    )(ts, wb, wb, keys)

    res = res.reshape(_B, _S, 1)
    weights = jnp.ones_like(res)
    return (res, weights)


# speedup vs baseline: 1.2472x; 1.0002x over previous
import jax
import jax.numpy as jnp
from jax import lax
from jax.experimental import pallas as pl
from jax.experimental.pallas import tpu as pltpu

_B = 16
_S = 512
_K = 10
_NUM_EXP = 500
_OVER = 5.0
_NB = 5
_DTMAX = 5.0
_R = _B * _S
_BLK_R = 256
_CHUNK = 128
_NCHUNK = 4
_WLANES = 128
_BIG = 3.0e38


def _softplus(z):
    return jnp.maximum(z, 0.0) + jnp.log1p(jnp.exp(-jnp.abs(z)))


_ROT = ((13, 15, 26, 6), (17, 29, 16, 24))


def _threefry_hash(k0, k1, x1):
    ks2 = k0 ^ k1 ^ jnp.uint32(0x1BD11BDA)
    inj = ((k1, ks2, 1), (ks2, k0, 2), (k0, k1, 3), (k1, ks2, 4), (ks2, k0, 5))
    x1 = x1 + k1
    x0 = x1 + k0
    r = _ROT[0][0]
    x1 = (x1 << jnp.uint32(r)) | (x1 >> jnp.uint32(32 - r))
    x1 = x0 ^ x1
    first = True
    for g in range(5):
        for r in _ROT[g % 2][(1 if first else 0):]:
            x0 = x0 + x1
            x1 = (x1 << jnp.uint32(r)) | (x1 >> jnp.uint32(32 - r))
            x1 = x0 ^ x1
        first = False
        a, b, c = inj[g]
        x0 = x0 + a
        x1 = x1 + (b + jnp.uint32(c))
    return x0 ^ x1


def _bits_to_unif(bits):
    f = lax.bitcast_convert_type((bits >> jnp.uint32(9)) | jnp.uint32(0x3F800000),
                                 jnp.float32)
    return f - 1.0


def _body(ts_ref, wbv_ref, wbs_ref, keys_ref, out_ref,
          ntri_ref, idxb_ref, bsc_ref):
    i = pl.program_id(0)

    @pl.when(i == 0)
    def _prep():
        wv = wbv_ref[0:1, :]
        bv = wbv_ref[1:2, :]
        subf = lax.broadcasted_iota(jnp.int32, (8, _WLANES), 0).astype(jnp.float32)
        lane = lax.broadcasted_iota(jnp.int32, (8, _WLANES), 1)
        steps = 1e-05 + subf * ((_DTMAX - 1e-05) / (_NB - 1))
        lam = _softplus(bv - wv * steps)
        lam = jnp.where(lane < _K, lam, 0.0)
        row = jnp.sum(lam, axis=1, keepdims=True)
        subi = lax.broadcasted_iota(jnp.int32, (8, 1), 0)
        row = jnp.where(subi < _NB, row, -jnp.inf)
        bound_v = jnp.max(row) * _OVER
        bsc_ref[0, 0] = bound_v
        bsc_ref[0, 1] = 1.0 / (bound_v + 1e-09)
        ntri_ref[...] = -(lax.broadcasted_iota(jnp.int32, (_CHUNK, _CHUNK), 0)
                          <= lax.broadcasted_iota(jnp.int32, (_CHUNK, _CHUNK), 1)
                          ).astype(jnp.float32)
        idxb_ref[...] = (lax.broadcasted_iota(jnp.int32, (_BLK_R, _CHUNK), 0)
                         * _NUM_EXP
                         + lax.broadcasted_iota(jnp.int32, (_BLK_R, _CHUNK), 1))

    bound = bsc_ref[0, 0]
    inv_bound = bsc_ref[0, 1]

    ke0 = keys_ref[0, 0]
    ke1 = keys_ref[0, 1]
    ku0 = keys_ref[0, 2]
    ku1 = keys_ref[0, 3]

    idxb = idxb_ref[...]
    ntri = ntri_ref[...]
    col_iota = lax.broadcasted_iota(jnp.int32, (_BLK_R, _CHUNK), 1)
    step_off = i * (_BLK_R * _NUM_EXP)

    def cond(state):
        j, carry, val = state
        return (j < _NCHUNK) & jnp.logical_not(jnp.all(val < _BIG))

    _LOG2E = 1.4426950408889634
    sk = [(wbs_ref[0, k] * inv_bound) * _LOG2E for k in range(_K)]
    bk = [wbs_ref[1, k] * _LOG2E for k in range(_K)]
    bound2 = bound * _LOG2E

    def chunk_body(state):
        j, carry, val = state
        colg = j * _CHUNK + col_iota
        idx = (idxb + (step_off + j * _CHUNK)).astype(jnp.uint32)
        u = _bits_to_unif(_threefry_hash(ke0, ke1, idx))
        unif = _bits_to_unif(_threefry_hash(ku0, ku1, idx))
        lanemask = colg < _NUM_EXP
        x = jnp.where(lanemask, jnp.log(u + 1e-09), 0.0)
        cs = jnp.dot(x, ntri, preferred_element_type=jnp.float32)
        c_un = carry + cs
        prod = jnp.ones_like(c_un)
        for k in range(_K):
            prod = prod * (1.0 + jnp.exp2(bk[k] - sk[k] * c_un))
        acc = (jnp.exp2(unif * bound2) < prod) & lanemask
        cmin = jnp.min(jnp.where(acc, c_un, _BIG), axis=1, keepdims=True)
        return j + 1, carry + cs[:, _CHUNK - 1:_CHUNK], jnp.minimum(val, cmin)

    idx0 = (idxb + step_off).astype(jnp.uint32)
    u0 = _bits_to_unif(_threefry_hash(ke0, ke1, idx0))
    unif0 = _bits_to_unif(_threefry_hash(ku0, ku1, idx0))
    x0 = jnp.log(u0 + 1e-09)
    cs0 = jnp.dot(x0, ntri, preferred_element_type=jnp.float32)
    prod0 = jnp.ones_like(cs0)
    for k in range(_K):
        prod0 = prod0 * (1.0 + jnp.exp2(bk[k] - sk[k] * cs0))
    acc0 = jnp.exp2(unif0 * bound2) < prod0
    val0 = jnp.min(jnp.where(acc0, cs0, _BIG), axis=1, keepdims=True)

    init = (jnp.int32(1), cs0[:, _CHUNK - 1:_CHUNK], val0)
    _, _, val = lax.while_loop(cond, chunk_body, init)

    res = jnp.where(val < _BIG,
                    jnp.minimum(ts_ref[...] + val * inv_bound, 100000.0),
                    jnp.float32(_DTMAX))
    out_ref[...] = res


def kernel(time_seq, time_delta_seq, event_seq, w, b):
    kroot = jax.random.key(1)
    ke, ku = jax.random.split(kroot)
    kde = jax.random.key_data(ke).astype(jnp.uint32)
    kdu = jax.random.key_data(ku).astype(jnp.uint32)
    keys = jnp.concatenate([kde, kdu]).reshape(1, 4)

    ts = time_seq.reshape(_R, 1)
    wb = jnp.zeros((2, _WLANES), jnp.float32)
    wb = wb.at[0, :_K].set(w).at[1, :_K].set(b)

    grid = (_R // _BLK_R,)
    res = pl.pallas_call(
        _body,
        grid=grid,
        in_specs=[
            pl.BlockSpec((_BLK_R, 1), lambda i: (i, 0)),
            pl.BlockSpec((2, _WLANES), lambda i: (0, 0)),
            pl.BlockSpec(memory_space=pltpu.SMEM),
            pl.BlockSpec(memory_space=pltpu.SMEM),
        ],
        out_specs=pl.BlockSpec((_BLK_R, 1), lambda i: (i, 0)),
        out_shape=jax.ShapeDtypeStruct((_R, 1), jnp.float32),
        scratch_shapes=[
            pltpu.VMEM((_CHUNK, _CHUNK), jnp.float32),
            pltpu.VMEM((_BLK_R, _CHUNK), jnp.int32),
            pltpu.SMEM((1, 2), jnp.float32),
        ],
    )(ts, wb, wb, keys)

    res = res.reshape(_B, _S, 1)
    weights = jnp.ones_like(res)
    return (res, weights)
